# baseline (speedup 1.0000x reference)
import functools

import jax
import jax.numpy as jnp
from jax import lax
from jax.experimental import pallas as pl
from jax.experimental.pallas import tpu as pltpu
from jax.experimental.pallas import tpu_sc as plsc

LANES = 16


@functools.lru_cache(maxsize=None)
def _build(batch, seqlen, embed, vocab):
    info = plsc.get_sparse_core_info()
    nc, ns = info.num_cores, info.num_subcores
    nw = nc * ns

    rows = batch * seqlen
    assert rows % nw == 0
    rows_per_w = rows // nw
    assert rows_per_w % seqlen == 0
    seq_per_chunk = 2
    chunk = seq_per_chunk * seqlen
    assert rows_per_w % chunk == 0
    nchunk = rows_per_w // chunk
    assert embed % LANES == 0
    evecs = embed // LANES

    mesh = plsc.VectorSubcoreMesh(core_axis_name="c", subcore_axis_name="s")

    @functools.partial(
        pl.kernel,
        mesh=mesh,
        compiler_params=pltpu.CompilerParams(use_tc_tiling_on_sc=False),
        out_type=jax.ShapeDtypeStruct((rows, embed), jnp.float32),
        scratch_types=[
            pltpu.VMEM((rows_per_w,), jnp.int32),
            pltpu.VMEM((seqlen, embed), jnp.float32),
            pltpu.VMEM((chunk, embed), jnp.float32),
            pltpu.SemaphoreType.DMA,
        ],
    )
    def emb(idx_hbm, table_hbm, pos_hbm, out_hbm, idx_v, pos_v, rows_v, sem):
        wid = lax.axis_index("s") * nc + lax.axis_index("c")
        base = wid * rows_per_w
        pltpu.sync_copy(idx_hbm.at[pl.ds(base, rows_per_w)], idx_v)
        pltpu.sync_copy(pos_hbm, pos_v)

        def chunk_body(g, carry):
            cbase = g * chunk
            pltpu.async_copy(
                table_hbm.at[idx_v.at[pl.ds(cbase, chunk)]], rows_v, sem
            ).wait()

            def row_body(r, carry2):
                for sq in range(seq_per_chunk):
                    for c in range(evecs):
                        pv = pos_v[r, pl.ds(c * LANES, LANES)]
                        plsc.addupdate(
                            rows_v.at[sq * seqlen + r, pl.ds(c * LANES, LANES)], pv
                        )
                return carry2

            lax.fori_loop(0, seqlen, row_body, 0)
            pltpu.sync_copy(rows_v, out_hbm.at[pl.ds(base + cbase, chunk)])
            return carry

        lax.fori_loop(0, nchunk, chunk_body, 0)

    return emb


def kernel(inputs, index_table, pos_table):
    batch, seqlen = inputs.shape
    vocab, embed = index_table.shape
    idx_flat = inputs.reshape(-1).astype(jnp.int32)
    emb = _build(batch, seqlen, embed, vocab)
    out = emb(idx_flat, index_table.astype(jnp.float32), pos_table.astype(jnp.float32))
    return out.reshape(batch, seqlen, embed)


# speedup vs baseline: 1.1094x; 1.1094x over previous
import functools

import jax
import jax.numpy as jnp
from jax import lax
from jax.experimental import pallas as pl
from jax.experimental.pallas import tpu as pltpu
from jax.experimental.pallas import tpu_sc as plsc

LANES = 16
NBUF = 4
PF = 2


@functools.lru_cache(maxsize=None)
def _build(batch, seqlen, embed, vocab):
    info = plsc.get_sparse_core_info()
    nc, ns = info.num_cores, info.num_subcores
    nw = nc * ns

    rows = batch * seqlen
    assert rows % nw == 0
    rows_per_w = rows // nw
    assert rows_per_w % seqlen == 0
    chunk = seqlen
    nchunk = rows_per_w // chunk
    assert nchunk % NBUF == 0 and nchunk >= 2 * NBUF
    assert embed % LANES == 0
    evecs = embed // LANES

    mesh = plsc.VectorSubcoreMesh(core_axis_name="c", subcore_axis_name="s")

    @functools.partial(
        pl.kernel,
        mesh=mesh,
        compiler_params=pltpu.CompilerParams(use_tc_tiling_on_sc=False),
        out_type=jax.ShapeDtypeStruct((rows, embed), jnp.float32),
        scratch_types=[
            pltpu.VMEM((rows_per_w,), jnp.int32),
            pltpu.VMEM((seqlen, embed), jnp.float32),
        ]
        + [pltpu.VMEM((chunk, embed), jnp.float32) for _ in range(NBUF)]
        + [pltpu.SemaphoreType.DMA for _ in range(2 * NBUF)],
    )
    def emb(idx_hbm, table_hbm, pos_hbm, out_hbm, idx_v, pos_v, *bufsem):
        bufs = bufsem[:NBUF]
        gsems = bufsem[NBUF : 2 * NBUF]
        wsems = bufsem[2 * NBUF :]
        wid = lax.axis_index("s") * nc + lax.axis_index("c")
        base = wid * rows_per_w
        pltpu.sync_copy(idx_hbm.at[pl.ds(base, rows_per_w)], idx_v)
        pltpu.sync_copy(pos_hbm, pos_v)

        def gather_desc(g, b):
            cb = g * chunk
            return pltpu.make_async_copy(
                table_hbm.at[idx_v.at[pl.ds(cb, chunk)]], bufs[b], gsems[b]
            )

        def write_desc(g, b):
            cb = g * chunk
            return pltpu.make_async_copy(
                bufs[b], out_hbm.at[pl.ds(base + cb, chunk)], wsems[b]
            )

        def add_pos(b):
            buf = bufs[b]

            def row_body(r, carry):
                for c in range(evecs):
                    sl = pl.ds(c * LANES, LANES)
                    plsc.addupdate(buf.at[r, sl], pos_v[r, sl])
                return carry

            lax.fori_loop(0, seqlen, row_body, 0, unroll=4)

        for g in range(PF):
            gather_desc(g, g % NBUF).start()
        for g in range(PF):
            b = g % NBUF
            gather_desc(g, b).wait()
            add_pos(b)
            write_desc(g, b).start()
            b2 = (g + PF) % NBUF
            gather_desc(g + PF, b2).start()

        def group_body(grp, carry):
            for j in range(NBUF):
                g = PF + grp * NBUF + j
                b = (PF + j) % NBUF
                gather_desc(g, b).wait()
                add_pos(b)
                write_desc(g, b).start()
                b2 = (PF + j + PF) % NBUF
                write_desc(g - PF, b2).wait()
                gather_desc(g + PF, b2).start()
            return carry

        lax.fori_loop(0, (nchunk - 2 * PF) // NBUF, group_body, 0)

        for g in range(nchunk - PF, nchunk):
            b = g % NBUF
            gather_desc(g, b).wait()
            add_pos(b)
            write_desc(g, b).start()
        for g in range(nchunk - NBUF, nchunk):
            write_desc(g, g % NBUF).wait()

    return emb


def kernel(inputs, index_table, pos_table):
    batch, seqlen = inputs.shape
    vocab, embed = index_table.shape
    idx_flat = inputs.reshape(-1).astype(jnp.int32)
    emb = _build(batch, seqlen, embed, vocab)
    out = emb(idx_flat, index_table.astype(jnp.float32), pos_table.astype(jnp.float32))
    return out.reshape(batch, seqlen, embed)
